# Initial kernel scaffold; baseline (speedup 1.0000x reference)
import jax
import jax.numpy as jnp
from jax.experimental import pallas as pl


def kernel(input):
    raise NotImplementedError("write your pallas kernel here")


# speedup vs baseline: 5.1946x; 5.1946x over previous
import functools

import jax
import jax.numpy as jnp
from jax import lax
from jax.experimental import pallas as pl
from jax.experimental.pallas import tpu as pltpu
from jax.experimental.pallas import tpu_sc as plsc

TOPK_K = 30
B, T, L = 64, 50, 8192
LANES = 16
NV = L // LANES
NBINS = 256
NUM_CORES = 2
NUM_SUBCORES = 16
NW = NUM_CORES * NUM_SUBCORES
B_PER_W = B // NW


def _zero_ref(ref, n_vregs, zero):
  def body(j, _):
    ref[pl.ds(j * LANES, LANES)] = zero
    return 0
  lax.fori_loop(0, n_vregs, body, 0)


def _suffix_scan(hist_ref, suffix_ref, need):
  def body(jj, st):
    carry_cnt, total_ge = st
    j = 15 - jj
    h = hist_ref[pl.ds(j * LANES, LANES)]
    rh = lax.rev(h, (0,))
    cs = plsc.cumsum(rh) + carry_cnt
    suffix_ref[pl.ds(j * LANES, LANES)] = lax.rev(cs, (0,))
    ge = (cs >= need).astype(jnp.int32)
    return carry_cnt + jnp.sum(h), total_ge + jnp.sum(ge)

  _, total_ge = lax.fori_loop(0, 16, body, (jnp.int32(0), jnp.int32(0)))
  d = total_ge - 1
  idx = jnp.minimum(d + 1, NBINS - 1)
  above = plsc.load_gather(suffix_ref, [jnp.full((LANES,), idx, jnp.int32)])
  count_above = jnp.where(d >= NBINS - 1, jnp.int32(0), jnp.max(above))
  return d, count_above


@jax.jit
def kernel(input):
  x_flat = input.reshape(-1)
  mesh = plsc.VectorSubcoreMesh(core_axis_name="c", subcore_axis_name="s")

  @functools.partial(
      pl.kernel,
      out_type=jax.ShapeDtypeStruct((B * T * L,), jnp.float32),
      mesh=mesh,
      scratch_types=[
          pltpu.VMEM((L,), jnp.float32),
          pltpu.VMEM((L,), jnp.float32),
          pltpu.VMEM((L,), jnp.float32),
          pltpu.VMEM((L,), jnp.int32),
          pltpu.VMEM((NBINS,), jnp.int32),
          pltpu.VMEM((NBINS,), jnp.int32),
      ],
      compiler_params=pltpu.CompilerParams(needs_layout_passes=False),
  )
  def k(x_hbm, o_hbm, in_v, carry_v, out_v, cand_v, hist_v, suffix_v):
    wid = lax.axis_index("s") * NUM_CORES + lax.axis_index("c")
    zero_f = jnp.zeros((LANES,), jnp.float32)
    zero_i = jnp.zeros((LANES,), jnp.int32)
    ones_i = jnp.ones((LANES,), jnp.int32)
    lane_iota = lax.iota(jnp.int32, LANES)

    for bi in range(B_PER_W):
      b = wid * B_PER_W + bi
      _zero_ref(carry_v, NV, zero_f)

      def t_body(t, _):
        row = (b * T + t) * L
        pltpu.sync_copy(x_hbm.at[pl.ds(row, L)], in_v)

        def add_body(j, mx):
          sl = pl.ds(j * LANES, LANES)
          nc = carry_v[sl] + in_v[sl]
          carry_v[sl] = nc
          return jnp.maximum(mx, nc)

        mx = lax.fori_loop(0, NV, add_body, zero_f)
        row_max = jnp.max(mx)
        mvec = jnp.full((LANES,), jnp.maximum(row_max, jnp.float32(1e-6)))
        r0 = plsc.bitcast(jnp.int32(0x7EF127EA) - plsc.bitcast(mvec, jnp.int32),
                          jnp.float32)
        r1 = r0 * (jnp.float32(2.0) - mvec * r0)
        scale = r1 * jnp.float32(NBINS)

        _zero_ref(hist_v, NBINS // LANES, zero_i)

        def hist_body(j, _):
          v = carry_v[pl.ds(j * LANES, LANES)]
          bins = jnp.minimum((v * scale).astype(jnp.int32), NBINS - 1)
          plsc.addupdate_scatter(hist_v, [bins], ones_i)
          return 0

        lax.fori_loop(0, NV, hist_body, 0)
        d, count_above = _suffix_scan(hist_v, suffix_v, jnp.int32(TOPK_K))

        def comp_body(j, off):
          v = carry_v[pl.ds(j * LANES, LANES)]
          bins = jnp.minimum((v * scale).astype(jnp.int32), NBINS - 1)
          m = bins == d
          mi = m.astype(jnp.int32)
          pos = off + plsc.cumsum(mi) - 1
          plsc.store_scatter(cand_v, [pos], plsc.bitcast(v, jnp.int32), mask=m)
          return off + jnp.sum(mi)

        c = lax.fori_loop(0, NV, comp_body, jnp.int32(0))
        nvc = lax.shift_right_logical(c + (LANES - 1), 4)

        need = jnp.int32(TOPK_K) - count_above
        prefix = jnp.int32(0)
        def _i32(x):
          return x - (1 << 32) if x >= (1 << 31) else x
        himasks = [0, _i32(0xFF000000), _i32(0xFFFF0000), _i32(0xFFFFFF00)]
        for shift, himask in zip((24, 16, 8, 0), himasks):
          _zero_ref(hist_v, NBINS // LANES, zero_i)

          def cb_body(j, _, shift=shift, himask=himask, prefix=prefix):
            bits = cand_v[pl.ds(j * LANES, LANES)]
            lane_ok = (j * LANES + lane_iota) < c
            pref_ok = (bits & jnp.int32(himask)) == prefix
            digit = lax.shift_right_logical(bits, jnp.int32(shift)) & (NBINS - 1)
            plsc.addupdate_scatter(hist_v, [digit], ones_i,
                                   mask=jnp.logical_and(lane_ok, pref_ok))
            return 0

          lax.fori_loop(0, nvc, cb_body, 0)
          dl, ca = _suffix_scan(hist_v, suffix_v, need)
          need = need - ca
          prefix = prefix | lax.shift_left(dl, jnp.int32(shift))

        theta = plsc.bitcast(jnp.full((LANES,), prefix, jnp.int32), jnp.float32)

        def out_body(j, _):
          sl = pl.ds(j * LANES, LANES)
          v = carry_v[sl]
          out_v[sl] = jnp.where(v >= theta, v, jnp.float32(0.0))
          return 0

        lax.fori_loop(0, NV, out_body, 0)
        pltpu.sync_copy(out_v, o_hbm.at[pl.ds(row, L)])
        return 0

      lax.fori_loop(0, T, t_body, 0)

  return k(x_flat).reshape(input.shape)


# speedup vs baseline: 8.6304x; 1.6614x over previous
import functools

import jax
import jax.numpy as jnp
from jax import lax
from jax.experimental import pallas as pl
from jax.experimental.pallas import tpu as pltpu
from jax.experimental.pallas import tpu_sc as plsc

TOPK_K = 30
B, T, L = 64, 50, 8192
LANES = 16
NV = L // LANES
UNROLL = 4
NUM_CORES = 2
NUM_SUBCORES = 16
NW = NUM_CORES * NUM_SUBCORES
B_PER_W = B // NW

RADIX_SHIFTS = (28, 24, 20, 16, 12, 8, 4, 0)


def _i32(x):
  return x - (1 << 32) if x >= (1 << 31) else x


HIMASKS = [_i32((0xFFFFFFFF << (s + 4)) & 0xFFFFFFFF) for s in RADIX_SHIFTS]


@jax.jit
def kernel(input):
  x_flat = input.reshape(-1)
  mesh = plsc.VectorSubcoreMesh(core_axis_name="c", subcore_axis_name="s")

  @functools.partial(
      pl.kernel,
      out_type=jax.ShapeDtypeStruct((B * T * L,), jnp.float32),
      mesh=mesh,
      scratch_types=[
          pltpu.VMEM((L,), jnp.float32),
          pltpu.VMEM((L,), jnp.float32),
          pltpu.VMEM((L,), jnp.float32),
          pltpu.VMEM((L,), jnp.int32),
          pltpu.VMEM((L,), jnp.int32),
          pltpu.VMEM((LANES,), jnp.int32),
          pltpu.VMEM((LANES,), jnp.int32),
      ],
      compiler_params=pltpu.CompilerParams(needs_layout_passes=False),
  )
  def k(x_hbm, o_hbm, in_v, carry_v, out_v, cand_v, cidx_v, hist_v, suf_v):
    wid = lax.axis_index("s") * NUM_CORES + lax.axis_index("c")
    zero_f = jnp.zeros((LANES,), jnp.float32)
    zero_i = jnp.zeros((LANES,), jnp.int32)
    ones_i = jnp.ones((LANES,), jnp.int32)
    lane_iota = lax.iota(jnp.int32, LANES)

    for bi in range(B_PER_W):
      b = wid * B_PER_W + bi

      def zc_body(j, _):
        for u in range(UNROLL):
          carry_v[pl.ds((j * UNROLL + u) * LANES, LANES)] = zero_f
        return 0

      lax.fori_loop(0, NV // UNROLL, zc_body, 0)

      def t_body(t, theta):
        row = (b * T + t) * L
        pltpu.sync_copy(x_hbm.at[pl.ds(row, L)], in_v)

        def acc_body(j, off_vec):
          for u in range(UNROLL):
            base = (j * UNROLL + u) * LANES
            sl = pl.ds(base, LANES)
            cv = carry_v[sl] + in_v[sl]
            carry_v[sl] = cv
            m = cv >= theta
            out_v[sl] = jnp.where(m, cv, jnp.float32(0.0))
            pos = off_vec + plsc.cumsum(m.astype(jnp.int32)) - 1
            plsc.store_scatter(cand_v, [pos], plsc.bitcast(cv, jnp.int32),
                               mask=m)
            plsc.store_scatter(cidx_v, [pos], lane_iota + base, mask=m)
            off_vec = off_vec + plsc.all_reduce_population_count(m)
          return off_vec

        c_vec = lax.fori_loop(0, NV // UNROLL, acc_body, zero_i)
        c = jnp.max(c_vec)
        nvc = lax.shift_right_logical(c + (LANES - 1), 4)

        need_vec = jnp.full((LANES,), TOPK_K, jnp.int32)
        prefix_vec = zero_i
        for shift, himask in zip(RADIX_SHIFTS, HIMASKS):
          hist_v[pl.ds(0, LANES)] = zero_i

          def fill_body(j, _, shift=shift, himask=himask,
                        prefix_vec=prefix_vec):
            bits = cand_v[pl.ds(j * LANES, LANES)]
            lane_ok = (j * LANES + lane_iota) < c_vec
            pref_ok = (bits & jnp.int32(himask)) == prefix_vec
            digit = lax.shift_right_logical(bits, jnp.int32(shift)) & 0xF
            plsc.addupdate_scatter(hist_v, [digit], ones_i,
                                   mask=jnp.logical_and(lane_ok, pref_ok))
            return 0

          lax.fori_loop(0, nvc, fill_body, 0)
          h = hist_v[pl.ds(0, LANES)]
          cs = plsc.cumsum(lax.rev(h, (0,)))
          suf_v[pl.ds(0, LANES)] = lax.rev(cs, (0,))
          d_vec = plsc.all_reduce_population_count(cs >= need_vec) - 1
          idx = jnp.minimum(d_vec + 1, LANES - 1)
          above = plsc.load_gather(suf_v, [idx])
          count_above = jnp.where(d_vec >= LANES - 1, zero_i, above)
          need_vec = need_vec - count_above
          prefix_vec = prefix_vec | lax.shift_left(d_vec, jnp.int32(shift))

        def corr_body(j, _):
          bits = cand_v[pl.ds(j * LANES, LANES)]
          idxs = cidx_v[pl.ds(j * LANES, LANES)]
          lane_ok = (j * LANES + lane_iota) < c_vec
          bad = jnp.logical_and(lane_ok, bits < prefix_vec)
          plsc.store_scatter(out_v, [idxs], zero_f, mask=bad)
          return 0

        lax.fori_loop(0, nvc, corr_body, 0)

        pltpu.sync_copy(out_v, o_hbm.at[pl.ds(row, L)])
        return plsc.bitcast(prefix_vec, jnp.float32)

      lax.fori_loop(0, T, t_body, zero_f)

  return k(x_flat).reshape(input.shape)
